# Initial kernel scaffold; baseline (speedup 1.0000x reference)
import jax
import jax.numpy as jnp
from jax.experimental import pallas as pl


def kernel(x, edge_index, edge_attr, batch, W1, b1, W2, b2, Wl, bl):
    raise NotImplementedError("write your pallas kernel here")


# speedup vs baseline: 23.9622x; 23.9622x over previous
import functools

import jax
import jax.numpy as jnp
from jax import lax
from jax.experimental import pallas as pl
from jax.experimental.pallas import tpu as pltpu
from jax.experimental.pallas import tpu_sc as plsc

NN = 10000
EE = 320000
GG = 128
DH = 128
DO = 64

NC, NS = 2, 16
NW = NC * NS
EW = EE // NW
CH = 80
NJ = EW // CH

NP = 10240
NPW = NP // NW
NJB = NPW // CH
RT = NP // NS
GC = 136
UW = GG * NP
UT = UW // NS
UZ = 4096
RNODE = 1024

_mesh = plsc.VectorSubcoreMesh(core_axis_name="c", subcore_axis_name="s")
_sc_params = pltpu.CompilerParams(needs_layout_passes=False, use_tc_tiling_on_sc=False)


@functools.partial(
    pl.kernel,
    out_type=(
        jax.ShapeDtypeStruct((NC, NP), jnp.float32),
        jax.ShapeDtypeStruct((NC, GC), jnp.float32),
    ),
    mesh=_mesh,
    compiler_params=_sc_params,
    scratch_types=[
        pltpu.VMEM((NJ, CH), jnp.int32),
        pltpu.VMEM((NJB, CH), jnp.int32),
        pltpu.VMEM((RT,), jnp.float32),
        pltpu.VMEM((CH,), jnp.float32),
        pltpu.VMEM_SHARED((NP,), jnp.float32),
        pltpu.VMEM_SHARED((GC,), jnp.float32),
    ],
)
def _sc_degree(dst_h, batch_h, deg_out, cnt_out, dstv, bv, zb, onev, deg_s, cnt_s):
    c = lax.axis_index("c")
    s = lax.axis_index("s")
    wid = s * NC + c

    def zstep(i, carry):
        zb[pl.ds(i * 16, 16)] = jnp.zeros((16,), jnp.float32)
        return carry

    lax.fori_loop(0, RT // 16, zstep, 0)
    for k in range(CH // 16):
        onev[pl.ds(k * 16, 16)] = jnp.ones((16,), jnp.float32)

    pltpu.sync_copy(zb, deg_s.at[pl.ds(s * RT, RT)])

    @pl.when(s == 0)
    def _():
        pltpu.sync_copy(zb.at[pl.ds(0, GC)], cnt_s)

    pltpu.sync_copy(dst_h.at[wid], dstv)
    pltpu.sync_copy(batch_h.at[wid], bv)
    plsc.subcore_barrier()

    def estep(j, carry):
        pltpu.sync_copy(onev, deg_s.at[dstv.at[j]], add=True)
        return carry

    lax.fori_loop(0, NJ, estep, 0)

    def bstep(j, carry):
        pltpu.sync_copy(onev, cnt_s.at[bv.at[j]], add=True)
        return carry

    lax.fori_loop(0, NJB, bstep, 0)
    plsc.subcore_barrier()

    pltpu.sync_copy(deg_s.at[pl.ds(s * RT, RT)], deg_out.at[c, pl.ds(s * RT, RT)])

    @pl.when(s == 0)
    def _():
        pltpu.sync_copy(cnt_s, cnt_out.at[c])


@functools.partial(
    pl.kernel,
    out_type=jax.ShapeDtypeStruct((NC, NP, DH), jnp.float32),
    mesh=_mesh,
    compiler_params=_sc_params,
    scratch_types=[
        pltpu.VMEM((NJ, CH), jnp.int32),
        pltpu.VMEM((NJ, CH), jnp.int32),
        pltpu.VMEM((CH, DH), jnp.float32),
        pltpu.VMEM((64, DH), jnp.float32),
        pltpu.VMEM_SHARED((NP, DH), jnp.float32),
        pltpu.SemaphoreType.DMA,
    ],
)
def _sc_scatter_rows(y_h, src_h, dst_h, z_out, srcv, dstv, rows, zb, z_s, sem):
    c = lax.axis_index("c")
    s = lax.axis_index("s")
    wid = s * NC + c

    def zfill(i, carry):
        zb[i // 8, pl.ds((i % 8) * 16, 16)] = jnp.zeros((16,), jnp.float32)
        return carry

    lax.fori_loop(0, 64 * (DH // 16), zfill, 0)
    for t in range(RT // 64):
        pltpu.sync_copy(zb, z_s.at[pl.ds(s * RT + t * 64, 64)])

    pltpu.sync_copy(src_h.at[wid], srcv)
    pltpu.sync_copy(dst_h.at[wid], dstv)
    plsc.subcore_barrier()

    def estep(j, carry):
        pltpu.async_copy(y_h.at[srcv.at[j]], rows, sem).wait()
        pltpu.sync_copy(rows, z_s.at[dstv.at[j]], add=True)
        return carry

    lax.fori_loop(0, NJ, estep, 0)
    plsc.subcore_barrier()

    for t in range(RT // 128):
        pltpu.sync_copy(
            z_s.at[pl.ds(s * RT + t * 128, 128)],
            z_out.at[c, pl.ds(s * RT + t * 128, 128)],
        )


@functools.partial(
    pl.kernel,
    out_type=jax.ShapeDtypeStruct((NC, UW), jnp.float32),
    mesh=_mesh,
    compiler_params=_sc_params,
    scratch_types=[
        pltpu.VMEM((NJ, CH), jnp.int32),
        pltpu.VMEM((NJ, CH), jnp.int32),
        pltpu.VMEM((NP // 16, 16), jnp.float32),
        pltpu.VMEM((NP // 16, 16), jnp.int32),
        pltpu.VMEM((CH,), jnp.int32),
        pltpu.VMEM((CH,), jnp.float32),
        pltpu.VMEM((UZ,), jnp.float32),
        pltpu.VMEM_SHARED((UW,), jnp.float32),
    ],
)
def _sc_scatter_u(dinv_h, bn_h, src_h, dst_h, u_out, srcv, dstv, dinvv, bnv,
                  idxb, valb, zb, u_s):
    c = lax.axis_index("c")
    s = lax.axis_index("s")
    wid = s * NC + c

    def zfill(i, carry):
        zb[pl.ds(i * 16, 16)] = jnp.zeros((16,), jnp.float32)
        return carry

    lax.fori_loop(0, UZ // 16, zfill, 0)
    for t in range(UT // UZ):
        pltpu.sync_copy(zb, u_s.at[pl.ds(s * UT + t * UZ, UZ)])

    pltpu.sync_copy(src_h.at[wid], srcv)
    pltpu.sync_copy(dst_h.at[wid], dstv)
    pltpu.sync_copy(dinv_h, dinvv)
    pltpu.sync_copy(bn_h, bnv)
    plsc.subcore_barrier()

    def estep(j, carry):
        for k in range(CH // 16):
            d16 = dstv[j, pl.ds(k * 16, 16)]
            s16 = srcv[j, pl.ds(k * 16, 16)]
            dr = lax.shift_right_logical(d16, 4)
            dc = lax.bitwise_and(d16, 15)
            dv = plsc.load_gather(dinvv, [dr, dc])
            bn = plsc.load_gather(bnv, [dr, dc])
            idxb[pl.ds(k * 16, 16)] = bn + s16
            valb[pl.ds(k * 16, 16)] = dv
        pltpu.sync_copy(valb, u_s.at[idxb], add=True)
        return carry

    lax.fori_loop(0, NJ, estep, 0)
    plsc.subcore_barrier()

    for t in range(UT // UZ):
        pltpu.sync_copy(
            u_s.at[pl.ds(s * UT + t * UZ, UZ)],
            u_out.at[c, pl.ds(s * UT + t * UZ, UZ)],
        )


def _tc1_body(x_ref, w1_ref, deg_ref, batch_ref, y_ref, dinv_ref, bn_ref):
    deg = deg_ref[0] + deg_ref[1]
    dinv = jnp.where(deg > 0.0, lax.rsqrt(deg), 0.0)
    xw = jnp.dot(x_ref[...], w1_ref[...], preferred_element_type=jnp.float32)
    y_ref[...] = xw * dinv
    dinv_ref[...] = dinv
    bn_ref[...] = batch_ref[...] * NP


def _tc1(x_pad, W1, deg3, batch_col):
    return pl.pallas_call(
        _tc1_body,
        grid=(NP // RNODE,),
        in_specs=[
            pl.BlockSpec((RNODE, DH), lambda i: (i, 0)),
            pl.BlockSpec((DH, DH), lambda i: (0, 0)),
            pl.BlockSpec((NC, RNODE, 1), lambda i: (0, i, 0)),
            pl.BlockSpec((RNODE, 1), lambda i: (i, 0)),
        ],
        out_specs=[
            pl.BlockSpec((RNODE, DH), lambda i: (i, 0)),
            pl.BlockSpec((RNODE, 1), lambda i: (i, 0)),
            pl.BlockSpec((RNODE, 1), lambda i: (i, 0)),
        ],
        out_shape=[
            jax.ShapeDtypeStruct((NP, DH), jnp.float32),
            jax.ShapeDtypeStruct((NP, 1), jnp.float32),
            jax.ShapeDtypeStruct((NP, 1), jnp.int32),
        ],
    )(x_pad, W1, deg3, batch_col)


def _tc2_body(z_ref, dinv_ref, u_ref, ct_ref, w2_ref, wl_ref, b1_ref, b2_ref,
              bl_ref, out_ref, acc):
    i = pl.program_id(0)

    @pl.when(i == 0)
    def _():
        acc[...] = jnp.zeros_like(acc)

    z = z_ref[0] + z_ref[1]
    dinv = dinv_ref[...]
    h = jnp.maximum(z * dinv + b1_ref[...], 0.0)
    hp = h * dinv
    u = u_ref[0] + u_ref[1]
    acc[...] += jnp.dot(u, hp, preferred_element_type=jnp.float32)

    @pl.when(i == pl.num_programs(0) - 1)
    def _():
        w2l = jnp.dot(w2_ref[...], wl_ref[...], preferred_element_type=jnp.float32)
        t = acc[...] / ct_ref[...]
        out_ref[...] = (
            jnp.dot(t, w2l, preferred_element_type=jnp.float32)
            + jnp.dot(b2_ref[...], wl_ref[...], preferred_element_type=jnp.float32)
            + bl_ref[...]
        )


def _tc2(z_p, dinv_col, u3, ct_col, W2, Wl, b1r, b2r, blr):
    return pl.pallas_call(
        _tc2_body,
        grid=(NP // RNODE,),
        in_specs=[
            pl.BlockSpec((NC, RNODE, DH), lambda i: (0, i, 0)),
            pl.BlockSpec((RNODE, 1), lambda i: (i, 0)),
            pl.BlockSpec((NC, GG, RNODE), lambda i: (0, 0, i)),
            pl.BlockSpec((GG, 1), lambda i: (0, 0)),
            pl.BlockSpec((DH, DH), lambda i: (0, 0)),
            pl.BlockSpec((DH, DO), lambda i: (0, 0)),
            pl.BlockSpec((1, DH), lambda i: (0, 0)),
            pl.BlockSpec((1, DH), lambda i: (0, 0)),
            pl.BlockSpec((1, DO), lambda i: (0, 0)),
        ],
        out_specs=pl.BlockSpec((GG, DO), lambda i: (0, 0)),
        out_shape=jax.ShapeDtypeStruct((GG, DO), jnp.float32),
        scratch_shapes=[pltpu.VMEM((GG, DH), jnp.float32)],
    )(z_p, dinv_col, u3, ct_col, W2, Wl, b1r, b2r, blr)


@jax.jit
def kernel(x, edge_index, edge_attr, batch, W1, b1, W2, b2, Wl, bl):
    del edge_attr
    src3 = edge_index[0].reshape(NW, NJ, CH)
    dst3 = edge_index[1].reshape(NW, NJ, CH)
    batch_p = jnp.concatenate(
        [batch, jnp.full((NP - NN,), GG, dtype=jnp.int32)]
    )
    x_pad = jnp.concatenate(
        [x, jnp.zeros((NP - NN, DH), dtype=jnp.float32)], axis=0
    )

    deg_p, cnt_p = _sc_degree(dst3, batch_p.reshape(NW, NJB, CH))
    y, dinv_col, bn_col = _tc1(
        x_pad, W1, deg_p.reshape(NC, NP, 1), batch_p.reshape(NP, 1)
    )
    z_p = _sc_scatter_rows(y, src3, dst3)
    u_p = _sc_scatter_u(
        dinv_col.reshape(NP // 16, 16), bn_col.reshape(NP // 16, 16), src3, dst3
    )

    cnt = jnp.clip(cnt_p[0, :GG] + cnt_p[1, :GG], 1.0).reshape(GG, 1)
    logits = _tc2(
        z_p,
        dinv_col,
        u_p.reshape(NC, GG, NP),
        cnt,
        W2,
        Wl,
        b1.reshape(1, DH),
        b2.reshape(1, DH),
        bl.reshape(1, DO),
    )
    return logits


# speedup vs baseline: 32.0372x; 1.3370x over previous
import functools

import jax
import jax.numpy as jnp
from jax import lax
from jax.experimental import pallas as pl
from jax.experimental.pallas import tpu as pltpu
from jax.experimental.pallas import tpu_sc as plsc

NN = 10000
EE = 320000
GG = 128
DH = 128
DO = 64

NC, NS = 2, 16
NW = NC * NS
EW = EE // NW
CH = 80
NJ = EW // CH

NP = 10240
NPW = NP // NW
NJB = NPW // CH
RT = NP // NS
GC = 136
UW = GG * NP
UT = UW // NS
UZ = 4096
RNODE = 1024

_mesh = plsc.VectorSubcoreMesh(core_axis_name="c", subcore_axis_name="s")
_sc_params = pltpu.CompilerParams(needs_layout_passes=False, use_tc_tiling_on_sc=False)


@functools.partial(
    pl.kernel,
    out_type=(
        jax.ShapeDtypeStruct((NC, NP), jnp.float32),
        jax.ShapeDtypeStruct((NC, GC), jnp.float32),
    ),
    mesh=_mesh,
    compiler_params=_sc_params,
    scratch_types=[
        pltpu.VMEM((NJ, CH), jnp.int32),
        pltpu.VMEM((NJB, CH), jnp.int32),
        pltpu.VMEM((RT,), jnp.float32),
        pltpu.VMEM((CH,), jnp.float32),
        pltpu.VMEM_SHARED((NP,), jnp.float32),
        pltpu.VMEM_SHARED((GC,), jnp.float32),
    ],
)
def _sc_degree(dst_h, batch_h, deg_out, cnt_out, dstv, bv, zb, onev, deg_s, cnt_s):
    c = lax.axis_index("c")
    s = lax.axis_index("s")
    wid = s * NC + c

    def zstep(i, carry):
        zb[pl.ds(i * 16, 16)] = jnp.zeros((16,), jnp.float32)
        return carry

    lax.fori_loop(0, RT // 16, zstep, 0)
    for k in range(CH // 16):
        onev[pl.ds(k * 16, 16)] = jnp.ones((16,), jnp.float32)

    pltpu.sync_copy(zb, deg_s.at[pl.ds(s * RT, RT)])

    @pl.when(s == 0)
    def _():
        pltpu.sync_copy(zb.at[pl.ds(0, GC)], cnt_s)

    pltpu.sync_copy(dst_h.at[wid], dstv)
    pltpu.sync_copy(batch_h.at[wid], bv)
    plsc.subcore_barrier()

    def estep(j, carry):
        pltpu.sync_copy(onev, deg_s.at[dstv.at[j]], add=True)
        return carry

    lax.fori_loop(0, NJ, estep, 0)

    def bstep(j, carry):
        pltpu.sync_copy(onev, cnt_s.at[bv.at[j]], add=True)
        return carry

    lax.fori_loop(0, NJB, bstep, 0)
    plsc.subcore_barrier()

    pltpu.sync_copy(deg_s.at[pl.ds(s * RT, RT)], deg_out.at[c, pl.ds(s * RT, RT)])

    @pl.when(s == 0)
    def _():
        pltpu.sync_copy(cnt_s, cnt_out.at[c])


@functools.partial(
    pl.kernel,
    out_type=jax.ShapeDtypeStruct((NC, NP, DH), jnp.float32),
    mesh=_mesh,
    compiler_params=_sc_params,
    scratch_types=[
        pltpu.VMEM((NJ, CH), jnp.int32),
        pltpu.VMEM((NJ, CH), jnp.int32),
        pltpu.VMEM((2, CH, DH), jnp.float32),
        pltpu.VMEM((32, DH), jnp.float32),
        pltpu.VMEM_SHARED((NP, DH), jnp.float32),
        pltpu.SemaphoreType.DMA((2,)),
    ],
)
def _sc_scatter_rows(y_h, src_h, dst_h, z_out, srcv, dstv, rows, zb, z_s, sems):
    c = lax.axis_index("c")
    s = lax.axis_index("s")
    wid = s * NC + c

    def zfill(i, carry):
        zb[i // 8, pl.ds((i % 8) * 16, 16)] = jnp.zeros((16,), jnp.float32)
        return carry

    lax.fori_loop(0, 32 * (DH // 16), zfill, 0)
    for t in range(RT // 32):
        pltpu.sync_copy(zb, z_s.at[pl.ds(s * RT + t * 32, 32)])

    pltpu.sync_copy(src_h.at[wid], srcv)
    pltpu.sync_copy(dst_h.at[wid], dstv)
    plsc.subcore_barrier()

    pltpu.async_copy(y_h.at[srcv.at[0]], rows.at[0], sems.at[0])

    def estep(j, carry):
        b = lax.rem(j, 2)
        nb = lax.rem(j + 1, 2)

        @pl.when(j + 1 < NJ)
        def _():
            pltpu.async_copy(y_h.at[srcv.at[j + 1]], rows.at[nb], sems.at[nb])

        pltpu.make_async_copy(y_h.at[srcv.at[j]], rows.at[b], sems.at[b]).wait()
        pltpu.sync_copy(rows.at[b], z_s.at[dstv.at[j]], add=True)
        return carry

    lax.fori_loop(0, NJ, estep, 0)
    plsc.subcore_barrier()

    for t in range(RT // 128):
        pltpu.sync_copy(
            z_s.at[pl.ds(s * RT + t * 128, 128)],
            z_out.at[c, pl.ds(s * RT + t * 128, 128)],
        )


@functools.partial(
    pl.kernel,
    out_type=jax.ShapeDtypeStruct((NC, UW), jnp.float32),
    mesh=_mesh,
    compiler_params=_sc_params,
    scratch_types=[
        pltpu.VMEM((NJ, CH), jnp.int32),
        pltpu.VMEM((NJ, CH), jnp.int32),
        pltpu.VMEM((NP // 16, 16), jnp.float32),
        pltpu.VMEM((NP // 16, 16), jnp.int32),
        pltpu.VMEM((CH,), jnp.int32),
        pltpu.VMEM((CH,), jnp.float32),
        pltpu.VMEM((UZ,), jnp.float32),
        pltpu.VMEM_SHARED((UW,), jnp.float32),
    ],
)
def _sc_scatter_u(dinv_h, bn_h, src_h, dst_h, u_out, srcv, dstv, dinvv, bnv,
                  idxb, valb, zb, u_s):
    c = lax.axis_index("c")
    s = lax.axis_index("s")
    wid = s * NC + c

    def zfill(i, carry):
        zb[pl.ds(i * 16, 16)] = jnp.zeros((16,), jnp.float32)
        return carry

    lax.fori_loop(0, UZ // 16, zfill, 0)
    for t in range(UT // UZ):
        pltpu.sync_copy(zb, u_s.at[pl.ds(s * UT + t * UZ, UZ)])

    pltpu.sync_copy(src_h.at[wid], srcv)
    pltpu.sync_copy(dst_h.at[wid], dstv)
    pltpu.sync_copy(dinv_h, dinvv)
    pltpu.sync_copy(bn_h, bnv)
    plsc.subcore_barrier()

    def estep(j, carry):
        for k in range(CH // 16):
            d16 = dstv[j, pl.ds(k * 16, 16)]
            s16 = srcv[j, pl.ds(k * 16, 16)]
            dr = lax.shift_right_logical(d16, 4)
            dc = lax.bitwise_and(d16, 15)
            dv = plsc.load_gather(dinvv, [dr, dc])
            bn = plsc.load_gather(bnv, [dr, dc])
            idxb[pl.ds(k * 16, 16)] = bn + s16
            valb[pl.ds(k * 16, 16)] = dv
        pltpu.sync_copy(valb, u_s.at[idxb], add=True)
        return carry

    lax.fori_loop(0, NJ, estep, 0)
    plsc.subcore_barrier()

    for t in range(UT // UZ):
        pltpu.sync_copy(
            u_s.at[pl.ds(s * UT + t * UZ, UZ)],
            u_out.at[c, pl.ds(s * UT + t * UZ, UZ)],
        )


def _tc1_body(x_ref, w1_ref, deg_ref, batch_ref, y_ref, dinv_ref, bn_ref):
    deg = deg_ref[0] + deg_ref[1]
    dinv = jnp.where(deg > 0.0, lax.rsqrt(deg), 0.0)
    xw = jnp.dot(x_ref[...], w1_ref[...], preferred_element_type=jnp.float32)
    y_ref[...] = xw * dinv
    dinv_ref[...] = dinv
    bn_ref[...] = batch_ref[...] * NP


def _tc1(x_pad, W1, deg3, batch_col):
    return pl.pallas_call(
        _tc1_body,
        grid=(NP // RNODE,),
        in_specs=[
            pl.BlockSpec((RNODE, DH), lambda i: (i, 0)),
            pl.BlockSpec((DH, DH), lambda i: (0, 0)),
            pl.BlockSpec((NC, RNODE, 1), lambda i: (0, i, 0)),
            pl.BlockSpec((RNODE, 1), lambda i: (i, 0)),
        ],
        out_specs=[
            pl.BlockSpec((RNODE, DH), lambda i: (i, 0)),
            pl.BlockSpec((RNODE, 1), lambda i: (i, 0)),
            pl.BlockSpec((RNODE, 1), lambda i: (i, 0)),
        ],
        out_shape=[
            jax.ShapeDtypeStruct((NP, DH), jnp.float32),
            jax.ShapeDtypeStruct((NP, 1), jnp.float32),
            jax.ShapeDtypeStruct((NP, 1), jnp.int32),
        ],
    )(x_pad, W1, deg3, batch_col)


def _tc2_body(z_ref, dinv_ref, u_ref, ct_ref, w2_ref, wl_ref, b1_ref, b2_ref,
              bl_ref, out_ref, acc):
    i = pl.program_id(0)

    @pl.when(i == 0)
    def _():
        acc[...] = jnp.zeros_like(acc)

    z = z_ref[0] + z_ref[1]
    dinv = dinv_ref[...]
    h = jnp.maximum(z * dinv + b1_ref[...], 0.0)
    hp = h * dinv
    u = u_ref[0] + u_ref[1]
    acc[...] += jnp.dot(u, hp, preferred_element_type=jnp.float32)

    @pl.when(i == pl.num_programs(0) - 1)
    def _():
        w2l = jnp.dot(w2_ref[...], wl_ref[...], preferred_element_type=jnp.float32)
        t = acc[...] / ct_ref[...]
        out_ref[...] = (
            jnp.dot(t, w2l, preferred_element_type=jnp.float32)
            + jnp.dot(b2_ref[...], wl_ref[...], preferred_element_type=jnp.float32)
            + bl_ref[...]
        )


def _tc2(z_p, dinv_col, u3, ct_col, W2, Wl, b1r, b2r, blr):
    return pl.pallas_call(
        _tc2_body,
        grid=(NP // RNODE,),
        in_specs=[
            pl.BlockSpec((NC, RNODE, DH), lambda i: (0, i, 0)),
            pl.BlockSpec((RNODE, 1), lambda i: (i, 0)),
            pl.BlockSpec((NC, GG, RNODE), lambda i: (0, 0, i)),
            pl.BlockSpec((GG, 1), lambda i: (0, 0)),
            pl.BlockSpec((DH, DH), lambda i: (0, 0)),
            pl.BlockSpec((DH, DO), lambda i: (0, 0)),
            pl.BlockSpec((1, DH), lambda i: (0, 0)),
            pl.BlockSpec((1, DH), lambda i: (0, 0)),
            pl.BlockSpec((1, DO), lambda i: (0, 0)),
        ],
        out_specs=pl.BlockSpec((GG, DO), lambda i: (0, 0)),
        out_shape=jax.ShapeDtypeStruct((GG, DO), jnp.float32),
        scratch_shapes=[pltpu.VMEM((GG, DH), jnp.float32)],
    )(z_p, dinv_col, u3, ct_col, W2, Wl, b1r, b2r, blr)


@jax.jit
def kernel(x, edge_index, edge_attr, batch, W1, b1, W2, b2, Wl, bl):
    del edge_attr
    src3 = edge_index[0].reshape(NW, NJ, CH)
    dst3 = edge_index[1].reshape(NW, NJ, CH)
    batch_p = jnp.concatenate(
        [batch, jnp.full((NP - NN,), GG, dtype=jnp.int32)]
    )
    x_pad = jnp.concatenate(
        [x, jnp.zeros((NP - NN, DH), dtype=jnp.float32)], axis=0
    )

    deg_p, cnt_p = _sc_degree(dst3, batch_p.reshape(NW, NJB, CH))
    y, dinv_col, bn_col = _tc1(
        x_pad, W1, deg_p.reshape(NC, NP, 1), batch_p.reshape(NP, 1)
    )
    z_p = _sc_scatter_rows(y, src3, dst3)
    u_p = _sc_scatter_u(
        dinv_col.reshape(NP // 16, 16), bn_col.reshape(NP // 16, 16), src3, dst3
    )

    cnt = jnp.clip(cnt_p[0, :GG] + cnt_p[1, :GG], 1.0).reshape(GG, 1)
    logits = _tc2(
        z_p,
        dinv_col,
        u_p.reshape(NC, GG, NP),
        cnt,
        W2,
        Wl,
        b1.reshape(1, DH),
        b2.reshape(1, DH),
        bl.reshape(1, DO),
    )
    return logits


# speedup vs baseline: 33.0895x; 1.0328x over previous
import functools

import jax
import jax.numpy as jnp
from jax import lax
from jax.experimental import pallas as pl
from jax.experimental.pallas import tpu as pltpu
from jax.experimental.pallas import tpu_sc as plsc

NN = 10000
EE = 320000
GG = 128
DH = 128
DO = 64

NC, NS = 2, 16
NW = NC * NS
EW = EE // NW
CH = 80
NJ = EW // CH

NP = 10240
NPW = NP // NW
NJB = NPW // CH
RT = NP // NS
GC = 136
UW = GG * NP
UT = UW // NS
UZ = 4096
RNODE = 1024

_mesh = plsc.VectorSubcoreMesh(core_axis_name="c", subcore_axis_name="s")
_sc_params = pltpu.CompilerParams(needs_layout_passes=False, use_tc_tiling_on_sc=False)


@functools.partial(
    pl.kernel,
    out_type=(
        jax.ShapeDtypeStruct((NC, NP), jnp.float32),
        jax.ShapeDtypeStruct((NC, GC), jnp.float32),
    ),
    mesh=_mesh,
    compiler_params=_sc_params,
    scratch_types=[
        pltpu.VMEM((NJ, CH), jnp.int32),
        pltpu.VMEM((NJB, CH), jnp.int32),
        pltpu.VMEM((RT,), jnp.float32),
        pltpu.VMEM((CH,), jnp.float32),
        pltpu.VMEM_SHARED((NP,), jnp.float32),
        pltpu.VMEM_SHARED((GC,), jnp.float32),
    ],
)
def _sc_degree(dst_h, batch_h, deg_out, cnt_out, dstv, bv, zb, onev, deg_s, cnt_s):
    c = lax.axis_index("c")
    s = lax.axis_index("s")
    wid = s * NC + c

    def zstep(i, carry):
        zb[pl.ds(i * 16, 16)] = jnp.zeros((16,), jnp.float32)
        return carry

    lax.fori_loop(0, RT // 16, zstep, 0)
    for k in range(CH // 16):
        onev[pl.ds(k * 16, 16)] = jnp.ones((16,), jnp.float32)

    pltpu.sync_copy(zb, deg_s.at[pl.ds(s * RT, RT)])

    @pl.when(s == 0)
    def _():
        pltpu.sync_copy(zb.at[pl.ds(0, GC)], cnt_s)

    pltpu.sync_copy(dst_h.at[wid], dstv)
    pltpu.sync_copy(batch_h.at[wid], bv)
    plsc.subcore_barrier()

    def estep(j, carry):
        pltpu.sync_copy(onev, deg_s.at[dstv.at[j]], add=True)
        return carry

    lax.fori_loop(0, NJ, estep, 0)

    def bstep(j, carry):
        pltpu.sync_copy(onev, cnt_s.at[bv.at[j]], add=True)
        return carry

    lax.fori_loop(0, NJB, bstep, 0)
    plsc.subcore_barrier()

    pltpu.sync_copy(deg_s.at[pl.ds(s * RT, RT)], deg_out.at[c, pl.ds(s * RT, RT)])

    @pl.when(s == 0)
    def _():
        pltpu.sync_copy(cnt_s, cnt_out.at[c])


@functools.partial(
    pl.kernel,
    out_type=jax.ShapeDtypeStruct((NC, NP, DH), jnp.float32),
    mesh=_mesh,
    compiler_params=_sc_params,
    scratch_types=[
        pltpu.VMEM((NJ, CH), jnp.int32),
        pltpu.VMEM((NJ, CH), jnp.int32),
        pltpu.VMEM((2, CH, DH), jnp.float32),
        pltpu.VMEM((32, DH), jnp.float32),
        pltpu.VMEM_SHARED((NP, DH), jnp.float32),
        pltpu.SemaphoreType.DMA((2,)),
    ],
)
def _sc_scatter_rows(y_h, src_h, dst_h, z_out, srcv, dstv, rows, zb, z_s, sems):
    c = lax.axis_index("c")
    s = lax.axis_index("s")
    wid = s * NC + c

    def zfill(i, carry):
        zb[i // 8, pl.ds((i % 8) * 16, 16)] = jnp.zeros((16,), jnp.float32)
        return carry

    lax.fori_loop(0, 32 * (DH // 16), zfill, 0)
    for t in range(RT // 32):
        pltpu.sync_copy(zb, z_s.at[pl.ds(s * RT + t * 32, 32)])

    pltpu.sync_copy(src_h.at[wid], srcv)
    pltpu.sync_copy(dst_h.at[wid], dstv)
    plsc.subcore_barrier()

    pltpu.async_copy(y_h.at[srcv.at[0]], rows.at[0], sems.at[0])

    def estep(j, carry):
        b = lax.rem(j, 2)
        nb = lax.rem(j + 1, 2)

        @pl.when(j + 1 < NJ)
        def _():
            pltpu.async_copy(y_h.at[srcv.at[j + 1]], rows.at[nb], sems.at[nb])

        pltpu.make_async_copy(y_h.at[srcv.at[j]], rows.at[b], sems.at[b]).wait()
        pltpu.sync_copy(rows.at[b], z_s.at[dstv.at[j]], add=True)
        return carry

    lax.fori_loop(0, NJ, estep, 0)
    plsc.subcore_barrier()

    for t in range(RT // 128):
        pltpu.sync_copy(
            z_s.at[pl.ds(s * RT + t * 128, 128)],
            z_out.at[c, pl.ds(s * RT + t * 128, 128)],
        )


@functools.partial(
    pl.kernel,
    out_type=jax.ShapeDtypeStruct((NC, UW), jnp.float32),
    mesh=_mesh,
    compiler_params=_sc_params,
    scratch_types=[
        pltpu.VMEM((NJ, CH), jnp.int32),
        pltpu.VMEM((NJ, CH), jnp.int32),
        pltpu.VMEM((NP // 16, 16), jnp.float32),
        pltpu.VMEM((NP // 16, 16), jnp.int32),
        pltpu.VMEM((2, CH), jnp.int32),
        pltpu.VMEM((2, CH), jnp.float32),
        pltpu.VMEM((UZ,), jnp.float32),
        pltpu.VMEM_SHARED((UW,), jnp.float32),
        pltpu.SemaphoreType.DMA((2,)),
    ],
)
def _sc_scatter_u(dinv_h, bn_h, src_h, dst_h, u_out, srcv, dstv, dinvv, bnv,
                  idxb, valb, zb, u_s, ssem):
    c = lax.axis_index("c")
    s = lax.axis_index("s")
    wid = s * NC + c

    def zfill(i, carry):
        zb[pl.ds(i * 16, 16)] = jnp.zeros((16,), jnp.float32)
        return carry

    lax.fori_loop(0, UZ // 16, zfill, 0)
    for t in range(UT // UZ):
        pltpu.sync_copy(zb, u_s.at[pl.ds(s * UT + t * UZ, UZ)])

    pltpu.sync_copy(src_h.at[wid], srcv)
    pltpu.sync_copy(dst_h.at[wid], dstv)
    pltpu.sync_copy(dinv_h, dinvv)
    pltpu.sync_copy(bn_h, bnv)
    plsc.subcore_barrier()

    def estep(j, carry):
        b = lax.rem(j, 2)

        @pl.when(j >= 2)
        def _():
            pltpu.make_async_copy(valb.at[b], u_s.at[idxb.at[b]], ssem.at[b]).wait()

        for k in range(CH // 16):
            d16 = dstv[j, pl.ds(k * 16, 16)]
            s16 = srcv[j, pl.ds(k * 16, 16)]
            dr = lax.shift_right_logical(d16, 4)
            dc = lax.bitwise_and(d16, 15)
            dv = plsc.load_gather(dinvv, [dr, dc])
            bn = plsc.load_gather(bnv, [dr, dc])
            idxb[b, pl.ds(k * 16, 16)] = bn + s16
            valb[b, pl.ds(k * 16, 16)] = dv
        pltpu.async_copy(valb.at[b], u_s.at[idxb.at[b]], ssem.at[b], add=True)
        return carry

    lax.fori_loop(0, NJ, estep, 0)
    for b in range(2):
        pltpu.make_async_copy(valb.at[b], u_s.at[idxb.at[b]], ssem.at[b]).wait()
    plsc.subcore_barrier()

    for t in range(UT // UZ):
        pltpu.sync_copy(
            u_s.at[pl.ds(s * UT + t * UZ, UZ)],
            u_out.at[c, pl.ds(s * UT + t * UZ, UZ)],
        )


def _tc1_body(x_ref, w1_ref, deg_ref, batch_ref, y_ref, dinv_ref, bn_ref):
    deg = deg_ref[0] + deg_ref[1]
    dinv = jnp.where(deg > 0.0, lax.rsqrt(deg), 0.0)
    xw = jnp.dot(x_ref[...], w1_ref[...], preferred_element_type=jnp.float32)
    y_ref[...] = xw * dinv
    dinv_ref[...] = dinv
    bn_ref[...] = batch_ref[...] * NP


def _tc1(x_pad, W1, deg3, batch_col):
    return pl.pallas_call(
        _tc1_body,
        grid=(NP // RNODE,),
        in_specs=[
            pl.BlockSpec((RNODE, DH), lambda i: (i, 0)),
            pl.BlockSpec((DH, DH), lambda i: (0, 0)),
            pl.BlockSpec((NC, RNODE, 1), lambda i: (0, i, 0)),
            pl.BlockSpec((RNODE, 1), lambda i: (i, 0)),
        ],
        out_specs=[
            pl.BlockSpec((RNODE, DH), lambda i: (i, 0)),
            pl.BlockSpec((RNODE, 1), lambda i: (i, 0)),
            pl.BlockSpec((RNODE, 1), lambda i: (i, 0)),
        ],
        out_shape=[
            jax.ShapeDtypeStruct((NP, DH), jnp.float32),
            jax.ShapeDtypeStruct((NP, 1), jnp.float32),
            jax.ShapeDtypeStruct((NP, 1), jnp.int32),
        ],
    )(x_pad, W1, deg3, batch_col)


def _tc2_body(z_ref, dinv_ref, u_ref, ct_ref, w2_ref, wl_ref, b1_ref, b2_ref,
              bl_ref, out_ref, acc):
    i = pl.program_id(0)

    @pl.when(i == 0)
    def _():
        acc[...] = jnp.zeros_like(acc)

    z = z_ref[0] + z_ref[1]
    dinv = dinv_ref[...]
    h = jnp.maximum(z * dinv + b1_ref[...], 0.0)
    hp = h * dinv
    u = u_ref[0] + u_ref[1]
    acc[...] += jnp.dot(u, hp, preferred_element_type=jnp.float32)

    @pl.when(i == pl.num_programs(0) - 1)
    def _():
        w2l = jnp.dot(w2_ref[...], wl_ref[...], preferred_element_type=jnp.float32)
        t = acc[...] / ct_ref[...]
        out_ref[...] = (
            jnp.dot(t, w2l, preferred_element_type=jnp.float32)
            + jnp.dot(b2_ref[...], wl_ref[...], preferred_element_type=jnp.float32)
            + bl_ref[...]
        )


def _tc2(z_p, dinv_col, u3, ct_col, W2, Wl, b1r, b2r, blr):
    return pl.pallas_call(
        _tc2_body,
        grid=(NP // RNODE,),
        in_specs=[
            pl.BlockSpec((NC, RNODE, DH), lambda i: (0, i, 0)),
            pl.BlockSpec((RNODE, 1), lambda i: (i, 0)),
            pl.BlockSpec((NC, GG, RNODE), lambda i: (0, 0, i)),
            pl.BlockSpec((GG, 1), lambda i: (0, 0)),
            pl.BlockSpec((DH, DH), lambda i: (0, 0)),
            pl.BlockSpec((DH, DO), lambda i: (0, 0)),
            pl.BlockSpec((1, DH), lambda i: (0, 0)),
            pl.BlockSpec((1, DH), lambda i: (0, 0)),
            pl.BlockSpec((1, DO), lambda i: (0, 0)),
        ],
        out_specs=pl.BlockSpec((GG, DO), lambda i: (0, 0)),
        out_shape=jax.ShapeDtypeStruct((GG, DO), jnp.float32),
        scratch_shapes=[pltpu.VMEM((GG, DH), jnp.float32)],
    )(z_p, dinv_col, u3, ct_col, W2, Wl, b1r, b2r, blr)


@jax.jit
def kernel(x, edge_index, edge_attr, batch, W1, b1, W2, b2, Wl, bl):
    del edge_attr
    src3 = edge_index[0].reshape(NW, NJ, CH)
    dst3 = edge_index[1].reshape(NW, NJ, CH)
    batch_p = jnp.concatenate(
        [batch, jnp.full((NP - NN,), GG, dtype=jnp.int32)]
    )
    x_pad = jnp.concatenate(
        [x, jnp.zeros((NP - NN, DH), dtype=jnp.float32)], axis=0
    )

    deg_p, cnt_p = _sc_degree(dst3, batch_p.reshape(NW, NJB, CH))
    y, dinv_col, bn_col = _tc1(
        x_pad, W1, deg_p.reshape(NC, NP, 1), batch_p.reshape(NP, 1)
    )
    z_p = _sc_scatter_rows(y, src3, dst3)
    u_p = _sc_scatter_u(
        dinv_col.reshape(NP // 16, 16), bn_col.reshape(NP // 16, 16), src3, dst3
    )

    cnt = jnp.clip(cnt_p[0, :GG] + cnt_p[1, :GG], 1.0).reshape(GG, 1)
    logits = _tc2(
        z_p,
        dinv_col,
        u_p.reshape(NC, GG, NP),
        cnt,
        W2,
        Wl,
        b1.reshape(1, DH),
        b2.reshape(1, DH),
        bl.reshape(1, DO),
    )
    return logits


# speedup vs baseline: 37.5732x; 1.1355x over previous
import functools

import jax
import jax.numpy as jnp
from jax import lax
from jax.experimental import pallas as pl
from jax.experimental.pallas import tpu as pltpu
from jax.experimental.pallas import tpu_sc as plsc

NN = 10000
EE = 320000
GG = 128
DH = 128
DO = 64

NC, NS = 2, 16
NW = NC * NS
EW = EE // NW
CH = 80
NJ = EW // CH

NP = 10240
NPW = NP // NW
NJB = NPW // CH
RT = NP // NS
GC = 136
UW = GG * NP
UT = UW // NS
UZ = 4096
RNODE = 1024

_mesh = plsc.VectorSubcoreMesh(core_axis_name="c", subcore_axis_name="s")
_sc_params = pltpu.CompilerParams(needs_layout_passes=False, use_tc_tiling_on_sc=False)


@functools.partial(
    pl.kernel,
    out_type=(
        jax.ShapeDtypeStruct((NC, NP), jnp.float32),
        jax.ShapeDtypeStruct((NC, GC), jnp.float32),
    ),
    mesh=_mesh,
    compiler_params=_sc_params,
    scratch_types=[
        pltpu.VMEM((NJ, CH), jnp.int32),
        pltpu.VMEM((NJB, CH), jnp.int32),
        pltpu.VMEM((RT,), jnp.float32),
        pltpu.VMEM((CH,), jnp.float32),
        pltpu.VMEM_SHARED((NP,), jnp.float32),
        pltpu.VMEM_SHARED((GC,), jnp.float32),
    ],
)
def _sc_degree(dst_h, batch_h, deg_out, cnt_out, dstv, bv, zb, onev, deg_s, cnt_s):
    c = lax.axis_index("c")
    s = lax.axis_index("s")
    wid = s * NC + c

    def zstep(i, carry):
        zb[pl.ds(i * 16, 16)] = jnp.zeros((16,), jnp.float32)
        return carry

    lax.fori_loop(0, RT // 16, zstep, 0)
    for k in range(CH // 16):
        onev[pl.ds(k * 16, 16)] = jnp.ones((16,), jnp.float32)

    pltpu.sync_copy(zb, deg_s.at[pl.ds(s * RT, RT)])

    @pl.when(s == 0)
    def _():
        pltpu.sync_copy(zb.at[pl.ds(0, GC)], cnt_s)

    pltpu.sync_copy(dst_h.at[wid], dstv)
    pltpu.sync_copy(batch_h.at[wid], bv)
    plsc.subcore_barrier()

    def estep(j, carry):
        pltpu.sync_copy(onev, deg_s.at[dstv.at[j]], add=True)
        return carry

    lax.fori_loop(0, NJ, estep, 0)

    def bstep(j, carry):
        pltpu.sync_copy(onev, cnt_s.at[bv.at[j]], add=True)
        return carry

    lax.fori_loop(0, NJB, bstep, 0)
    plsc.subcore_barrier()

    pltpu.sync_copy(deg_s.at[pl.ds(s * RT, RT)], deg_out.at[c, pl.ds(s * RT, RT)])

    @pl.when(s == 0)
    def _():
        pltpu.sync_copy(cnt_s, cnt_out.at[c])


@functools.partial(
    pl.kernel,
    out_type=jax.ShapeDtypeStruct((NC, NP, DH), jnp.float32),
    mesh=_mesh,
    compiler_params=_sc_params,
    scratch_types=[
        pltpu.VMEM((NJ, CH), jnp.int32),
        pltpu.VMEM((NJ, CH), jnp.int32),
        pltpu.VMEM((2, CH, DH), jnp.float32),
        pltpu.VMEM((32, DH), jnp.float32),
        pltpu.VMEM_SHARED((NP, DH), jnp.float32),
        pltpu.SemaphoreType.DMA((2,)),
    ],
)
def _sc_scatter_rows(y_h, src_h, dst_h, z_out, srcv, dstv, rows, zb, z_s, sems):
    c = lax.axis_index("c")
    s = lax.axis_index("s")
    wid = s * NC + c

    def zfill(i, carry):
        zb[i // 8, pl.ds((i % 8) * 16, 16)] = jnp.zeros((16,), jnp.float32)
        return carry

    lax.fori_loop(0, 32 * (DH // 16), zfill, 0)
    for t in range(RT // 32):
        pltpu.sync_copy(zb, z_s.at[pl.ds(s * RT + t * 32, 32)])

    pltpu.sync_copy(src_h.at[wid], srcv)
    pltpu.sync_copy(dst_h.at[wid], dstv)
    plsc.subcore_barrier()

    pltpu.async_copy(y_h.at[srcv.at[0]], rows.at[0], sems.at[0])

    def estep(j, carry):
        b = lax.rem(j, 2)
        nb = lax.rem(j + 1, 2)

        @pl.when(j + 1 < NJ)
        def _():
            pltpu.async_copy(y_h.at[srcv.at[j + 1]], rows.at[nb], sems.at[nb])

        pltpu.make_async_copy(y_h.at[srcv.at[j]], rows.at[b], sems.at[b]).wait()
        pltpu.sync_copy(rows.at[b], z_s.at[dstv.at[j]], add=True)
        return carry

    lax.fori_loop(0, NJ, estep, 0)
    plsc.subcore_barrier()

    for t in range(RT // 128):
        pltpu.sync_copy(
            z_s.at[pl.ds(s * RT + t * 128, 128)],
            z_out.at[c, pl.ds(s * RT + t * 128, 128)],
        )


@functools.partial(
    pl.kernel,
    out_type=jax.ShapeDtypeStruct((NC, GG, NP), jnp.float32),
    mesh=_mesh,
    compiler_params=_sc_params,
    scratch_types=[
        pltpu.VMEM((NJ, CH), jnp.int32),
        pltpu.VMEM((NJ, CH), jnp.int32),
        pltpu.VMEM((NP,), jnp.float32),
        pltpu.VMEM((NP,), jnp.int32),
        pltpu.VMEM((2, CH), jnp.int32),
        pltpu.VMEM((2, CH), jnp.float32),
        pltpu.VMEM((UZ,), jnp.float32),
        pltpu.VMEM_SHARED((UW,), jnp.float32),
        pltpu.SemaphoreType.DMA((2,)),
    ],
)
def _sc_scatter_u(dinv_h, bn_h, src_h, dst_h, u_out, srcv, dstv, dinvv, bnv,
                  idxb, valb, zb, u_s, ssem):
    c = lax.axis_index("c")
    s = lax.axis_index("s")
    wid = s * NC + c

    def zfill(i, carry):
        zb[pl.ds(i * 16, 16)] = jnp.zeros((16,), jnp.float32)
        return carry

    lax.fori_loop(0, UZ // 16, zfill, 0)
    for t in range(UT // UZ):
        pltpu.sync_copy(zb, u_s.at[pl.ds(s * UT + t * UZ, UZ)])

    pltpu.sync_copy(src_h.at[wid], srcv)
    pltpu.sync_copy(dst_h.at[wid], dstv)
    pltpu.sync_copy(dinv_h.at[0], dinvv)
    pltpu.sync_copy(bn_h.at[0], bnv)
    plsc.subcore_barrier()

    def estep(j, carry):
        b = lax.rem(j, 2)

        @pl.when(j >= 2)
        def _():
            pltpu.make_async_copy(valb.at[b], u_s.at[idxb.at[b]], ssem.at[b]).wait()

        for k in range(CH // 16):
            d16 = dstv[j, pl.ds(k * 16, 16)]
            s16 = srcv[j, pl.ds(k * 16, 16)]
            dv = plsc.load_gather(dinvv, [d16])
            bn = plsc.load_gather(bnv, [d16])
            idxb[b, pl.ds(k * 16, 16)] = bn + s16
            valb[b, pl.ds(k * 16, 16)] = dv
        pltpu.async_copy(valb.at[b], u_s.at[idxb.at[b]], ssem.at[b], add=True)
        return carry

    lax.fori_loop(0, NJ, estep, 0)
    for b in range(2):
        pltpu.make_async_copy(valb.at[b], u_s.at[idxb.at[b]], ssem.at[b]).wait()
    plsc.subcore_barrier()

    for r in range(GG // NS):
        pltpu.sync_copy(
            u_s.at[pl.ds((s * (GG // NS) + r) * NP, NP)],
            u_out.at[c, s * (GG // NS) + r],
        )


def _tc1_body(x_ref, w1_ref, deg_ref, batch_ref, y_ref, dinv_ref, bn_ref):
    dsum = deg_ref[0:1, :] + deg_ref[1:2, :]
    dinv = jnp.where(dsum > 0.0, lax.rsqrt(dsum), 0.0)
    xw = jnp.dot(x_ref[...], w1_ref[...], preferred_element_type=jnp.float32)
    y_ref[...] = xw * jnp.reshape(dinv, (RNODE, 1))
    dinv_ref[...] = dinv
    bn_ref[...] = batch_ref[...] * NP


def _tc1(x_pad, W1, deg_p, batch_row):
    return pl.pallas_call(
        _tc1_body,
        grid=(NP // RNODE,),
        in_specs=[
            pl.BlockSpec((RNODE, DH), lambda i: (i, 0)),
            pl.BlockSpec((DH, DH), lambda i: (0, 0)),
            pl.BlockSpec((NC, RNODE), lambda i: (0, i)),
            pl.BlockSpec((1, RNODE), lambda i: (0, i)),
        ],
        out_specs=[
            pl.BlockSpec((RNODE, DH), lambda i: (i, 0)),
            pl.BlockSpec((1, RNODE), lambda i: (0, i)),
            pl.BlockSpec((1, RNODE), lambda i: (0, i)),
        ],
        out_shape=[
            jax.ShapeDtypeStruct((NP, DH), jnp.float32),
            jax.ShapeDtypeStruct((1, NP), jnp.float32),
            jax.ShapeDtypeStruct((1, NP), jnp.int32),
        ],
    )(x_pad, W1, deg_p, batch_row)


def _tc2_body(z_ref, dinv_ref, u_ref, ct_ref, w2_ref, wl_ref, b1_ref, b2_ref,
              bl_ref, out_ref, acc):
    i = pl.program_id(0)

    @pl.when(i == 0)
    def _():
        acc[...] = jnp.zeros_like(acc)

    z = z_ref[0] + z_ref[1]
    dinv = jnp.reshape(dinv_ref[...], (RNODE, 1))
    h = jnp.maximum(z * dinv + b1_ref[...], 0.0)
    hp = h * dinv
    u = u_ref[0] + u_ref[1]
    acc[...] += jnp.dot(u, hp, preferred_element_type=jnp.float32)

    @pl.when(i == pl.num_programs(0) - 1)
    def _():
        w2l = jnp.dot(w2_ref[...], wl_ref[...], preferred_element_type=jnp.float32)
        csum = ct_ref[0:1, :GG] + ct_ref[1:2, :GG]
        ct = jnp.maximum(jnp.reshape(csum, (GG, 1)), 1.0)
        t = acc[...] / ct
        out_ref[...] = (
            jnp.dot(t, w2l, preferred_element_type=jnp.float32)
            + jnp.dot(b2_ref[...], wl_ref[...], preferred_element_type=jnp.float32)
            + bl_ref[...]
        )


def _tc2(z_p, dinv_row, u3, cnt_p, W2, Wl, b1r, b2r, blr):
    return pl.pallas_call(
        _tc2_body,
        grid=(NP // RNODE,),
        in_specs=[
            pl.BlockSpec((NC, RNODE, DH), lambda i: (0, i, 0)),
            pl.BlockSpec((1, RNODE), lambda i: (0, i)),
            pl.BlockSpec((NC, GG, RNODE), lambda i: (0, 0, i)),
            pl.BlockSpec((NC, GC), lambda i: (0, 0)),
            pl.BlockSpec((DH, DH), lambda i: (0, 0)),
            pl.BlockSpec((DH, DO), lambda i: (0, 0)),
            pl.BlockSpec((1, DH), lambda i: (0, 0)),
            pl.BlockSpec((1, DH), lambda i: (0, 0)),
            pl.BlockSpec((1, DO), lambda i: (0, 0)),
        ],
        out_specs=pl.BlockSpec((GG, DO), lambda i: (0, 0)),
        out_shape=jax.ShapeDtypeStruct((GG, DO), jnp.float32),
        scratch_shapes=[pltpu.VMEM((GG, DH), jnp.float32)],
    )(z_p, dinv_row, u3, cnt_p, W2, Wl, b1r, b2r, blr)


@jax.jit
def kernel(x, edge_index, edge_attr, batch, W1, b1, W2, b2, Wl, bl):
    del edge_attr
    src3 = edge_index[0].reshape(NW, NJ, CH)
    dst3 = edge_index[1].reshape(NW, NJ, CH)
    batch_p = jnp.concatenate(
        [batch, jnp.full((NP - NN,), GG, dtype=jnp.int32)]
    )
    x_pad = jnp.concatenate(
        [x, jnp.zeros((NP - NN, DH), dtype=jnp.float32)], axis=0
    )

    deg_p, cnt_p = _sc_degree(dst3, batch_p.reshape(NW, NJB, CH))
    y, dinv_row, bn_row = _tc1(x_pad, W1, deg_p, batch_p.reshape(1, NP))
    z_p = _sc_scatter_rows(y, src3, dst3)
    u_p = _sc_scatter_u(dinv_row, bn_row, src3, dst3)
    logits = _tc2(
        z_p,
        dinv_row,
        u_p,
        cnt_p,
        W2,
        Wl,
        b1.reshape(1, DH),
        b2.reshape(1, DH),
        bl.reshape(1, DO),
    )
    return logits
